# Initial kernel scaffold; baseline (speedup 1.0000x reference)
import jax
import jax.numpy as jnp
from jax.experimental import pallas as pl


def kernel(x0_1, x1, x2, adj0_row, adj0_col, adj1_row, adj1_col, adj2_row, adj2_col, W1, b1, a1w, a1b, a2w, a2b, Wagg, bagg):
    raise NotImplementedError("write your pallas kernel here")


# speedup vs baseline: 12.8412x; 12.8412x over previous
import functools

import jax
import jax.numpy as jnp
from jax import lax
from jax.experimental import pallas as pl
from jax.experimental.pallas import tpu as pltpu
from jax.experimental.pallas import tpu_sc as plsc

N0 = 10000
N1 = 160000
N2 = 50000
E0 = 320000
E1 = 320000
E2 = 150000
D = 128
H = 4
YW = 4 * D + 16
ROWBLK = 400
C = 64
NW = 32
PADROW = N0
OUTPAD = 10240
A1PAD = N0 + 16

def _cdiv(a, b):
    return (a + b - 1) // b

NC0 = _cdiv(E0, NW * C)
NC1 = _cdiv(E1, NW * C)
NC2 = _cdiv(E2, NW * C)
E0P = NC0 * NW * C
E1P = NC1 * NW * C
E2P = NC2 * NW * C


def _mm_t(x, w):
    return lax.dot_general(x, w, (((1,), (1,)), ((), ())),
                           preferred_element_type=jnp.float32)


def _mm(x, w):
    return lax.dot_general(x, w, (((1,), (0,)), ((), ())),
                           preferred_element_type=jnp.float32)


def _head_body(x_ref, w1_ref, b1_ref, wagg_ref, bagg_ref, ma1_ref, ma2_ref,
               brows_ref, y_ref, a1_ref, base_ref, *, blk, with_base):
    x = x_ref[...]
    a2acc = jnp.broadcast_to(brows_ref[1:2, :], (ROWBLK, 16))
    if with_base:
        a1acc = jnp.broadcast_to(brows_ref[0:1, :], (ROWBLK, 16))
        base = jnp.zeros((ROWBLK, D), jnp.float32)
    for h in range(H):
        xh = jax.nn.relu(_mm_t(x, w1_ref[h]) + b1_ref[h][None, :])
        y_ref[:, h * D:(h + 1) * D] = 0.25 * _mm_t(
            x=xh, w=wagg_ref[h, :, blk * D:(blk + 1) * D])
        a2acc = a2acc + _mm(xh, ma2_ref[h])
        if with_base:
            base = base + 0.25 * (_mm_t(xh, wagg_ref[h, :, 0:D])
                                  + bagg_ref[h][None, :])
            a1acc = a1acc + _mm(xh, ma1_ref[h])
    y_ref[:, 4 * D:YW] = a2acc
    if with_base:
        a1_ref[...] = a1acc
        base_ref[...] = base


def _dense_call(x, w1, b1, wagg, bagg, ma1, ma2, brows, *, blk, with_base):
    n = x.shape[0]
    grid = (n // ROWBLK,)
    full = lambda shape: pl.BlockSpec(shape, lambda i: tuple(0 for _ in shape))
    in_specs = [
        pl.BlockSpec((ROWBLK, D), lambda i: (i, 0)),
        full((H, D, D)), full((H, D)), full((H, D, 4 * D)), full((H, D)),
        full((H, D, 16)), full((H, D, 16)), full((8, 16)),
    ]
    if with_base:
        out_shape = [
            jax.ShapeDtypeStruct((n, YW), jnp.float32),
            jax.ShapeDtypeStruct((n, 16), jnp.float32),
            jax.ShapeDtypeStruct((n, D), jnp.float32),
        ]
        out_specs = [
            pl.BlockSpec((ROWBLK, YW), lambda i: (i, 0)),
            pl.BlockSpec((ROWBLK, 16), lambda i: (i, 0)),
            pl.BlockSpec((ROWBLK, D), lambda i: (i, 0)),
        ]
        body = functools.partial(_head_body, blk=blk, with_base=True)
    else:
        out_shape = [jax.ShapeDtypeStruct((n, YW), jnp.float32)]
        out_specs = [pl.BlockSpec((ROWBLK, YW), lambda i: (i, 0))]

        def body(x_ref, w1_ref, b1_ref, wagg_ref, bagg_ref, ma1_ref, ma2_ref,
                 brows_ref, y_ref):
            _head_body(x_ref, w1_ref, b1_ref, wagg_ref, bagg_ref, ma1_ref,
                       ma2_ref, brows_ref, y_ref, None, None,
                       blk=blk, with_base=False)

    return pl.pallas_call(
        body, grid=grid, in_specs=in_specs, out_specs=out_specs,
        out_shape=out_shape,
    )(x, w1, b1, wagg, bagg, ma1, ma2, brows)


def _sc_edges(y0, y1, y2, a1t, r0, c0, r1, c1, r2, c2):
    mesh = plsc.VectorSubcoreMesh(core_axis_name="c", subcore_axis_name="s")

    @functools.partial(
        pl.kernel, mesh=mesh,
        out_type=jax.ShapeDtypeStruct((2, OUTPAD, D), jnp.float32),
        compiler_params=pltpu.CompilerParams(use_tc_tiling_on_sc=False),
        scratch_types=[
            pltpu.VMEM_SHARED((OUTPAD, D), jnp.float32),
            pltpu.VMEM((16, D), jnp.float32),
            pltpu.VMEM((C,), jnp.int32),
            pltpu.VMEM((C,), jnp.int32),
            pltpu.VMEM((C, 16), jnp.float32),
            pltpu.VMEM((C, YW), jnp.float32),
            pltpu.VMEM((C, D), jnp.float32),
            pltpu.SemaphoreType.DMA,
            pltpu.SemaphoreType.DMA,
        ],
    )
    def k(y0_hbm, y1_hbm, y2_hbm, a1_hbm, r0_hbm, c0_hbm, r1_hbm, c1_hbm,
          r2_hbm, c2_hbm, out_hbm, acc_sp, zbuf, rows_v, cols_v, a1c, yc,
          vout, sem_a, sem_y):
        cid = lax.axis_index("c")
        sid = lax.axis_index("s")
        g = cid * 16 + sid

        def zrow(i, _):
            for j in range(D // 16):
                zbuf[i, pl.ds(j * 16, 16)] = jnp.zeros((16,), jnp.float32)
            return 0
        lax.fori_loop(0, 16, zrow, 0)

        def zcp(i, _):
            pltpu.sync_copy(zbuf, acc_sp.at[pl.ds(sid * 640 + i * 16, 16)])
            return 0
        lax.fori_loop(0, 40, zcp, 0)
        plsc.subcore_barrier()

        def phase(r_hbm, c_hbm, y_hbm, nck):
            def chunk(i, _):
                base = pl.multiple_of((g * nck + i) * C, C)
                pltpu.sync_copy(r_hbm.at[pl.ds(base, C)], rows_v)
                pltpu.sync_copy(c_hbm.at[pl.ds(base, C)], cols_v)
                cpa = pltpu.async_copy(a1_hbm.at[rows_v], a1c, sem_a)
                cpy = pltpu.async_copy(y_hbm.at[cols_v], yc, sem_y)
                cpa.wait()
                cpy.wait()

                def edge(e, _):
                    av = a1c[e, :]
                    ys = yc[e, pl.ds(4 * D, 16)]
                    att = 1.0 / (1.0 + jnp.exp(-(av + ys)))
                    for j in range(D // 16):
                        acc = None
                        for h in range(H):
                            s = att[h]
                            v = yc[e, pl.ds(h * D + j * 16, 16)]
                            acc = v * s if acc is None else acc + v * s
                        vout[e, pl.ds(j * 16, 16)] = acc
                    return 0
                lax.fori_loop(0, C, edge, 0)
                pltpu.sync_copy(vout, acc_sp.at[rows_v], add=True)
                return 0
            lax.fori_loop(0, nck, chunk, 0)

        phase(r0_hbm, c0_hbm, y0_hbm, NC0)
        phase(r1_hbm, c1_hbm, y1_hbm, NC1)
        phase(r2_hbm, c2_hbm, y2_hbm, NC2)
        plsc.subcore_barrier()

        def wcp(i, _):
            off = sid * 640 + i * 64
            pltpu.sync_copy(acc_sp.at[pl.ds(off, 64)],
                            out_hbm.at[cid, pl.ds(off, 64)])
            return 0
        lax.fori_loop(0, 10, wcp, 0)

    return k(y0, y1, y2, a1t, r0, c0, r1, c1, r2, c2)


def kernel(x0_1, x1, x2, adj0_row, adj0_col, adj1_row, adj1_col,
           adj2_row, adj2_col, W1, b1, a1w, a1b, a2w, a2b, Wagg, bagg):
    f32 = jnp.float32
    i32 = jnp.int32

    ma1 = jnp.stack([jnp.zeros((D, 16), f32).at[:, h].set(a1w[h])
                     for h in range(H)])
    ma2 = jnp.stack([jnp.zeros((D, 16), f32).at[:, h].set(a2w[h])
                     for h in range(H)])
    brows = (jnp.zeros((8, 16), f32)
             .at[0, :H].set(a1b)
             .at[1, :H].set(a2b))

    y0, a1t, base = _dense_call(x0_1, W1, b1, Wagg, bagg, ma1, ma2, brows,
                                blk=1, with_base=True)
    (y1,) = _dense_call(x1, W1, b1, Wagg, bagg, ma1, ma2, brows,
                        blk=2, with_base=False)
    (y2,) = _dense_call(x2, W1, b1, Wagg, bagg, ma1, ma2, brows,
                        blk=3, with_base=False)

    a1p = jnp.concatenate([a1t, jnp.zeros((A1PAD - N0, 16), f32)])

    def pad_edges(rows, cols, ep):
        e = rows.shape[0]
        rp = jnp.concatenate([rows, jnp.full((ep - e,), PADROW, i32)])
        cp = jnp.concatenate([cols, jnp.zeros((ep - e,), i32)])
        return rp, cp

    r0, c0 = pad_edges(adj0_row, adj0_col, E0P)
    r1, c1 = pad_edges(adj1_row, adj1_col, E1P)
    r2, c2 = pad_edges(adj2_row, adj2_col, E2P)

    outp = _sc_edges(y0, y1, y2, a1p, r0, c0, r1, c1, r2, c2)
    return base + outp[0, :N0] + outp[1, :N0]
